# baseline (speedup 1.0000x reference)
import jax
import jax.numpy as jnp
from jax.experimental import pallas as pl
from jax.experimental.pallas import tpu as pltpu

_P = 7
_SCALE = 1.0 / 16.0
_NROW = 9


_BN = 8


def _pool_kernel(hrow_s, wrow_s, xt_ref, empty_ref, out_ref, u_ref):
    g = pl.program_id(0)

    def body(b, carry):
        base = (g * _BN + b) * (_P * _NROW)
        for ph in range(_P):
            o = base + ph * _NROW
            acc = xt_ref[hrow_s[o]]
            for j in range(1, _NROW):
                acc = jnp.maximum(acc, xt_ref[hrow_s[o + j]])
            u_ref[:, ph, :] = acc
        for pw in range(_P):
            o = base + pw * _NROW
            acc = u_ref[wrow_s[o]]
            for j in range(1, _NROW):
                acc = jnp.maximum(acc, u_ref[wrow_s[o + j]])
            e = empty_ref[b, :, pw : pw + 1]
            out_ref[b, :, pw, :] = jnp.where(e > 0.0, 0.0, acc)
        return carry

    jax.lax.fori_loop(0, _BN, body, 0)


def _roi_bins(rois, rois_index, H, W):
    n = rois.shape[0]
    boxes = rois[:, jnp.array([1, 0, 3, 2])]
    xs = jnp.round(boxes[:, 0] * _SCALE)
    ys = jnp.round(boxes[:, 1] * _SCALE)
    xe = jnp.round(boxes[:, 2] * _SCALE)
    ye = jnp.round(boxes[:, 3] * _SCALE)
    bw = jnp.maximum(xe - xs + 1.0, 1.0) / _P
    bh = jnp.maximum(ye - ys + 1.0, 1.0) / _P
    pbin = jnp.arange(_P, dtype=jnp.float32)
    ws = jnp.clip(jnp.floor(pbin[None, :] * bw[:, None]) + xs[:, None], 0, W).astype(jnp.int32)
    we = jnp.clip(jnp.ceil((pbin[None, :] + 1.0) * bw[:, None]) + xs[:, None], 0, W).astype(jnp.int32)
    hs = jnp.clip(jnp.floor(pbin[None, :] * bh[:, None]) + ys[:, None], 0, H).astype(jnp.int32)
    he = jnp.clip(jnp.ceil((pbin[None, :] + 1.0) * bh[:, None]) + ys[:, None], 0, H).astype(jnp.int32)
    j = jnp.arange(_NROW, dtype=jnp.int32)
    hrow = jnp.clip(hs[:, :, None] + jnp.minimum(j[None, None, :], (he - hs)[:, :, None] - 1), 0, H - 1)
    habs = rois_index[:, None, None] * H + hrow
    wrow = jnp.clip(ws[:, :, None] + jnp.minimum(j[None, None, :], (we - ws)[:, :, None] - 1), 0, W - 1)
    empty = ((hs >= he)[:, :, None] | (ws >= we)[:, None, :]).astype(jnp.float32)
    return habs.reshape(n * _P * _NROW), wrow.reshape(n * _P * _NROW), empty


def _pool(x, rois, rois_index):
    B, C, H, W = x.shape
    n = rois.shape[0]
    habs, wrow, empty = _roi_bins(rois, rois_index, H, W)
    xt = x.transpose(0, 2, 3, 1).reshape(B * H, W, C)
    return pl.pallas_call(
        _pool_kernel,
        out_shape=jax.ShapeDtypeStruct((n, _P, _P, C), x.dtype),
        grid_spec=pltpu.PrefetchScalarGridSpec(
            num_scalar_prefetch=2,
            grid=(n // _BN,),
            in_specs=[
                pl.BlockSpec((B * H, W, C), lambda g, hr, wr: (0, 0, 0)),
                pl.BlockSpec((_BN, _P, _P), lambda g, hr, wr: (g, 0, 0)),
            ],
            out_specs=pl.BlockSpec((_BN, _P, _P, C), lambda g, hr, wr: (g, 0, 0, 0)),
            scratch_shapes=[pltpu.VMEM((W, _P, C), x.dtype)],
        ),
        compiler_params=pltpu.CompilerParams(
            dimension_semantics=("parallel",),
            vmem_limit_bytes=56 * 1024 * 1024,
        ),
        name="roi_pool",
    )(habs, wrow, xt, empty)


def _fc6_kernel(h_ref, w_ref, b_ref, o_ref):
    k = pl.program_id(0)
    nk = pl.num_programs(0)

    @pl.when(k == 0)
    def _():
        o_ref[...] = jnp.zeros_like(o_ref)

    o_ref[...] += jnp.dot(h_ref[...], w_ref[...], preferred_element_type=jnp.float32)

    @pl.when(k == nk - 1)
    def _():
        o_ref[...] = jnp.maximum(o_ref[...] + b_ref[...], 0.0)


def _fc6(h0, W1, b1):
    n, K = h0.shape
    J = W1.shape[1]
    bk = 896
    return pl.pallas_call(
        _fc6_kernel,
        out_shape=jax.ShapeDtypeStruct((n, J), jnp.float32),
        grid=(K // bk,),
        in_specs=[
            pl.BlockSpec((n, bk), lambda k: (0, k)),
            pl.BlockSpec((bk, J), lambda k: (k, 0)),
            pl.BlockSpec((1, J), lambda k: (0, 0)),
        ],
        out_specs=pl.BlockSpec((n, J), lambda k: (0, 0)),
        compiler_params=pltpu.CompilerParams(
            dimension_semantics=("arbitrary",),
            vmem_limit_bytes=56 * 1024 * 1024,
        ),
        name="fc6",
    )(h0, W1, b1)


def _fc7_heads_kernel(h_ref, w2_ref, b2_ref, whc_ref, bhc_ref, o_ref):
    j = pl.program_id(0)
    nj = pl.num_programs(0)
    t = jnp.maximum(
        jnp.dot(h_ref[...], w2_ref[...], preferred_element_type=jnp.float32) + b2_ref[...], 0.0
    )

    @pl.when(j == 0)
    def _():
        o_ref[...] = bhc_ref[...] + jnp.zeros_like(o_ref)

    o_ref[...] += jnp.dot(t, whc_ref[...], preferred_element_type=jnp.float32)


def _fc7_heads(h1, W2, b2, whc, bhc):
    n, K = h1.shape
    M = whc.shape[1]
    bj = 512
    return pl.pallas_call(
        _fc7_heads_kernel,
        out_shape=jax.ShapeDtypeStruct((n, M), jnp.float32),
        grid=(K // bj,),
        in_specs=[
            pl.BlockSpec((n, K), lambda j: (0, 0)),
            pl.BlockSpec((K, bj), lambda j: (0, j)),
            pl.BlockSpec((1, bj), lambda j: (0, j)),
            pl.BlockSpec((bj, M), lambda j: (j, 0)),
            pl.BlockSpec((1, M), lambda j: (0, 0)),
        ],
        out_specs=pl.BlockSpec((n, M), lambda j: (0, 0)),
        compiler_params=pltpu.CompilerParams(
            dimension_semantics=("arbitrary",),
            vmem_limit_bytes=56 * 1024 * 1024,
        ),
        name="fc7_heads",
    )(h1, W2, b2, whc, bhc)


def kernel(x, rois, rois_index, W1, b1, W2, b2, Wb, bb, Wc, bc):
    B, C, H, W = x.shape
    n = rois.shape[0]
    pooled = _pool(x, rois, rois_index)
    h0 = pooled.transpose(0, 3, 1, 2).reshape(n, C * _P * _P)
    h1 = _fc6(h0, W1, b1.reshape(1, -1))
    whc = jnp.concatenate([Wb, Wc], axis=1)
    bhc = jnp.concatenate([bb, bc]).reshape(1, -1)
    heads = _fc7_heads(h1, W2, b2.reshape(1, -1), whc, bhc)
    nb = Wb.shape[1]
    return heads[:, :nb], heads[:, nb:]


# speedup vs baseline: 1.1964x; 1.1964x over previous
import jax
import jax.numpy as jnp
from jax.experimental import pallas as pl
from jax.experimental.pallas import tpu as pltpu

_P = 7
_SCALE = 1.0 / 16.0
_NROW = 9


_BN = 8


_NA = 3


def _pool_kernel(hrow_s, wrow_s, xt_hbm, empty_ref, out_ref, a2_ref, u_ref, sem):
    g = pl.program_id(0)
    BH = xt_hbm.shape[0]
    H = u_ref.shape[0]

    @pl.when(g == 0)
    def _build():
        cp = pltpu.make_async_copy(xt_hbm, a2_ref.at[pl.ds(0, BH)], sem)
        cp.start()
        cp.wait()
        for img in range(BH // H):
            b0 = img * H
            t = BH + b0
            a2_ref[pl.ds(t, H - 1)] = jnp.maximum(
                a2_ref[pl.ds(b0, H - 1)], a2_ref[pl.ds(b0 + 1, H - 1)]
            )
            a2_ref[t + H - 1] = a2_ref[b0 + H - 1]
            a2_ref[pl.ds(t, H - 2)] = jnp.maximum(
                a2_ref[pl.ds(t, H - 2)], a2_ref[pl.ds(t + 2, H - 2)]
            )

    def body(b, carry):
        baseA = (g * _BN + b) * (_P * _NA)
        baseB = (g * _BN + b) * (_P * _NROW)
        for ph in range(_P):
            o = baseA + ph * _NA
            acc = a2_ref[hrow_s[o]]
            for j in range(1, _NA):
                acc = jnp.maximum(acc, a2_ref[hrow_s[o + j]])
            u_ref[:, ph, :] = acc
        for pw in range(_P):
            o = baseB + pw * _NROW
            acc = u_ref[wrow_s[o]]
            for j in range(1, _NROW):
                acc = jnp.maximum(acc, u_ref[wrow_s[o + j]])
            e = empty_ref[b, :, pw : pw + 1]
            out_ref[b, :, pw, :] = jnp.where(e > 0.0, 0.0, acc)
        return carry

    jax.lax.fori_loop(0, _BN, body, 0)


def _roi_bins(rois, rois_index, B, H, W):
    n = rois.shape[0]
    boxes = rois[:, jnp.array([1, 0, 3, 2])]
    xs = jnp.round(boxes[:, 0] * _SCALE)
    ys = jnp.round(boxes[:, 1] * _SCALE)
    xe = jnp.round(boxes[:, 2] * _SCALE)
    ye = jnp.round(boxes[:, 3] * _SCALE)
    bw = jnp.maximum(xe - xs + 1.0, 1.0) / _P
    bh = jnp.maximum(ye - ys + 1.0, 1.0) / _P
    pbin = jnp.arange(_P, dtype=jnp.float32)
    ws = jnp.clip(jnp.floor(pbin[None, :] * bw[:, None]) + xs[:, None], 0, W).astype(jnp.int32)
    we = jnp.clip(jnp.ceil((pbin[None, :] + 1.0) * bw[:, None]) + xs[:, None], 0, W).astype(jnp.int32)
    hs = jnp.clip(jnp.floor(pbin[None, :] * bh[:, None]) + ys[:, None], 0, H).astype(jnp.int32)
    he = jnp.clip(jnp.ceil((pbin[None, :] + 1.0) * bh[:, None]) + ys[:, None], 0, H).astype(jnp.int32)
    j = jnp.arange(_NROW, dtype=jnp.int32)
    wrow = jnp.clip(ws[:, :, None] + jnp.minimum(j[None, None, :], (we - ws)[:, :, None] - 1), 0, W - 1)
    empty = ((hs >= he)[:, :, None] | (ws >= we)[:, None, :]).astype(jnp.float32)
    BH = B * H
    d = he - hs
    hs_c = jnp.clip(hs, 0, H - 1)
    e1 = jnp.maximum(jnp.clip(he - 1, 0, H - 1), hs_c)
    small = d <= 3
    r0 = jnp.where(small, hs_c, BH + hs)
    r1 = jnp.where(small, jnp.minimum(hs_c + 1, e1), BH + he - 4)
    r2 = jnp.where(small, jnp.minimum(hs_c + 2, e1), BH + jnp.where(d >= 9, hs + 2, hs))
    hrow3 = jnp.stack([r0, r1, r2], axis=-1)
    habs = rois_index[:, None, None] * H + hrow3
    return habs.reshape(n * _P * _NA), wrow.reshape(n * _P * _NROW), empty


def _pool(x, rois, rois_index):
    B, C, H, W = x.shape
    n = rois.shape[0]
    habs, wrow, empty = _roi_bins(rois, rois_index, B, H, W)
    xt = x.transpose(0, 2, 3, 1).reshape(B * H, W, C)
    return pl.pallas_call(
        _pool_kernel,
        out_shape=jax.ShapeDtypeStruct((n, _P, _P, C), x.dtype),
        grid_spec=pltpu.PrefetchScalarGridSpec(
            num_scalar_prefetch=2,
            grid=(n // _BN,),
            in_specs=[
                pl.BlockSpec(memory_space=pl.ANY),
                pl.BlockSpec((_BN, _P, _P), lambda g, hr, wr: (g, 0, 0)),
            ],
            out_specs=pl.BlockSpec((_BN, _P, _P, C), lambda g, hr, wr: (g, 0, 0, 0)),
            scratch_shapes=[
                pltpu.VMEM((2 * B * H, W, C), x.dtype),
                pltpu.VMEM((W, _P, C), x.dtype),
                pltpu.SemaphoreType.DMA,
            ],
        ),
        compiler_params=pltpu.CompilerParams(
            dimension_semantics=("arbitrary",),
            vmem_limit_bytes=56 * 1024 * 1024,
        ),
        name="roi_pool",
    )(habs, wrow, xt, empty)


def _fc6_kernel(h_ref, w_ref, b_ref, o_ref):
    k = pl.program_id(0)
    nk = pl.num_programs(0)

    @pl.when(k == 0)
    def _():
        o_ref[...] = jnp.zeros_like(o_ref)

    o_ref[...] += jnp.dot(h_ref[...], w_ref[...], preferred_element_type=jnp.float32)

    @pl.when(k == nk - 1)
    def _():
        o_ref[...] = jnp.maximum(o_ref[...] + b_ref[...], 0.0)


def _fc6(h0, W1, b1):
    n, K = h0.shape
    J = W1.shape[1]
    bk = 896
    return pl.pallas_call(
        _fc6_kernel,
        out_shape=jax.ShapeDtypeStruct((n, J), jnp.float32),
        grid=(K // bk,),
        in_specs=[
            pl.BlockSpec((n, bk), lambda k: (0, k)),
            pl.BlockSpec((bk, J), lambda k: (k, 0)),
            pl.BlockSpec((1, J), lambda k: (0, 0)),
        ],
        out_specs=pl.BlockSpec((n, J), lambda k: (0, 0)),
        compiler_params=pltpu.CompilerParams(
            dimension_semantics=("arbitrary",),
            vmem_limit_bytes=56 * 1024 * 1024,
        ),
        name="fc6",
    )(h0, W1, b1)


def _fc7_heads_kernel(h_ref, w2_ref, b2_ref, whc_ref, bhc_ref, o_ref):
    j = pl.program_id(0)
    nj = pl.num_programs(0)
    t = jnp.maximum(
        jnp.dot(h_ref[...], w2_ref[...], preferred_element_type=jnp.float32) + b2_ref[...], 0.0
    )

    @pl.when(j == 0)
    def _():
        o_ref[...] = bhc_ref[...] + jnp.zeros_like(o_ref)

    o_ref[...] += jnp.dot(t, whc_ref[...], preferred_element_type=jnp.float32)


def _fc7_heads(h1, W2, b2, whc, bhc):
    n, K = h1.shape
    M = whc.shape[1]
    bj = 512
    return pl.pallas_call(
        _fc7_heads_kernel,
        out_shape=jax.ShapeDtypeStruct((n, M), jnp.float32),
        grid=(K // bj,),
        in_specs=[
            pl.BlockSpec((n, K), lambda j: (0, 0)),
            pl.BlockSpec((K, bj), lambda j: (0, j)),
            pl.BlockSpec((1, bj), lambda j: (0, j)),
            pl.BlockSpec((bj, M), lambda j: (j, 0)),
            pl.BlockSpec((1, M), lambda j: (0, 0)),
        ],
        out_specs=pl.BlockSpec((n, M), lambda j: (0, 0)),
        compiler_params=pltpu.CompilerParams(
            dimension_semantics=("arbitrary",),
            vmem_limit_bytes=56 * 1024 * 1024,
        ),
        name="fc7_heads",
    )(h1, W2, b2, whc, bhc)


def kernel(x, rois, rois_index, W1, b1, W2, b2, Wb, bb, Wc, bc):
    B, C, H, W = x.shape
    n = rois.shape[0]
    pooled = _pool(x, rois, rois_index)
    h0 = pooled.transpose(0, 3, 1, 2).reshape(n, C * _P * _P)
    h1 = _fc6(h0, W1, b1.reshape(1, -1))
    whc = jnp.concatenate([Wb, Wc], axis=1)
    bhc = jnp.concatenate([bb, bc]).reshape(1, -1)
    heads = _fc7_heads(h1, W2, b2.reshape(1, -1), whc, bhc)
    nb = Wb.shape[1]
    return heads[:, :nb], heads[:, nb:]


# speedup vs baseline: 1.2025x; 1.0050x over previous
import jax
import jax.numpy as jnp
from jax.experimental import pallas as pl
from jax.experimental.pallas import tpu as pltpu

_P = 7
_SCALE = 1.0 / 16.0
_NROW = 9


_BN = 16


_NA = 3


def _pool_kernel(hrow_s, wrow_s, xt_hbm, empty_ref, out_ref, a2_ref, u_ref, sem):
    g = pl.program_id(0)
    BH = xt_hbm.shape[0]
    H = u_ref.shape[0]

    @pl.when(g == 0)
    def _build():
        cp = pltpu.make_async_copy(xt_hbm, a2_ref.at[pl.ds(0, BH)], sem)
        cp.start()
        cp.wait()
        for img in range(BH // H):
            b0 = img * H
            t = BH + b0
            a2_ref[pl.ds(t, H - 1)] = jnp.maximum(
                a2_ref[pl.ds(b0, H - 1)], a2_ref[pl.ds(b0 + 1, H - 1)]
            )
            a2_ref[t + H - 1] = a2_ref[b0 + H - 1]
            a2_ref[pl.ds(t, H - 2)] = jnp.maximum(
                a2_ref[pl.ds(t, H - 2)], a2_ref[pl.ds(t + 2, H - 2)]
            )

    def body(b, carry):
        baseA = (g * _BN + b) * (_P * _NA)
        baseB = (g * _BN + b) * (_P * _NROW)
        for ph in range(_P):
            o = baseA + ph * _NA
            acc = a2_ref[hrow_s[o]]
            for j in range(1, _NA):
                acc = jnp.maximum(acc, a2_ref[hrow_s[o + j]])
            u_ref[:, ph, :] = acc
        for pw in range(_P):
            o = baseB + pw * _NROW
            acc = u_ref[wrow_s[o]]
            for j in range(1, _NROW):
                acc = jnp.maximum(acc, u_ref[wrow_s[o + j]])
            e = empty_ref[b, :, pw : pw + 1]
            out_ref[b, :, pw, :] = jnp.where(e > 0.0, 0.0, acc)
        return carry

    jax.lax.fori_loop(0, _BN, body, 0)


def _roi_bins(rois, rois_index, B, H, W):
    n = rois.shape[0]
    boxes = rois[:, jnp.array([1, 0, 3, 2])]
    xs = jnp.round(boxes[:, 0] * _SCALE)
    ys = jnp.round(boxes[:, 1] * _SCALE)
    xe = jnp.round(boxes[:, 2] * _SCALE)
    ye = jnp.round(boxes[:, 3] * _SCALE)
    bw = jnp.maximum(xe - xs + 1.0, 1.0) / _P
    bh = jnp.maximum(ye - ys + 1.0, 1.0) / _P
    pbin = jnp.arange(_P, dtype=jnp.float32)
    ws = jnp.clip(jnp.floor(pbin[None, :] * bw[:, None]) + xs[:, None], 0, W).astype(jnp.int32)
    we = jnp.clip(jnp.ceil((pbin[None, :] + 1.0) * bw[:, None]) + xs[:, None], 0, W).astype(jnp.int32)
    hs = jnp.clip(jnp.floor(pbin[None, :] * bh[:, None]) + ys[:, None], 0, H).astype(jnp.int32)
    he = jnp.clip(jnp.ceil((pbin[None, :] + 1.0) * bh[:, None]) + ys[:, None], 0, H).astype(jnp.int32)
    j = jnp.arange(_NROW, dtype=jnp.int32)
    wrow = jnp.clip(ws[:, :, None] + jnp.minimum(j[None, None, :], (we - ws)[:, :, None] - 1), 0, W - 1)
    empty = ((hs >= he)[:, :, None] | (ws >= we)[:, None, :]).astype(jnp.float32)
    BH = B * H
    d = he - hs
    hs_c = jnp.clip(hs, 0, H - 1)
    e1 = jnp.maximum(jnp.clip(he - 1, 0, H - 1), hs_c)
    small = d <= 3
    r0 = jnp.where(small, hs_c, BH + hs)
    r1 = jnp.where(small, jnp.minimum(hs_c + 1, e1), BH + he - 4)
    r2 = jnp.where(small, jnp.minimum(hs_c + 2, e1), BH + jnp.where(d >= 9, hs + 2, hs))
    hrow3 = jnp.stack([r0, r1, r2], axis=-1)
    habs = rois_index[:, None, None] * H + hrow3
    return habs.reshape(n * _P * _NA), wrow.reshape(n * _P * _NROW), empty


def _pool(x, rois, rois_index):
    B, C, H, W = x.shape
    n = rois.shape[0]
    habs, wrow, empty = _roi_bins(rois, rois_index, B, H, W)
    xt = x.transpose(0, 2, 3, 1).reshape(B * H, W, C)
    return pl.pallas_call(
        _pool_kernel,
        out_shape=jax.ShapeDtypeStruct((n, _P, _P, C), x.dtype),
        grid_spec=pltpu.PrefetchScalarGridSpec(
            num_scalar_prefetch=2,
            grid=(n // _BN,),
            in_specs=[
                pl.BlockSpec(memory_space=pl.ANY),
                pl.BlockSpec((_BN, _P, _P), lambda g, hr, wr: (g, 0, 0)),
            ],
            out_specs=pl.BlockSpec((_BN, _P, _P, C), lambda g, hr, wr: (g, 0, 0, 0)),
            scratch_shapes=[
                pltpu.VMEM((2 * B * H, W, C), x.dtype),
                pltpu.VMEM((W, _P, C), x.dtype),
                pltpu.SemaphoreType.DMA,
            ],
        ),
        compiler_params=pltpu.CompilerParams(
            dimension_semantics=("arbitrary",),
            vmem_limit_bytes=56 * 1024 * 1024,
        ),
        name="roi_pool",
    )(habs, wrow, xt, empty)


def _fc6_kernel(h_ref, w_ref, b_ref, o_ref):
    k = pl.program_id(0)
    nk = pl.num_programs(0)

    @pl.when(k == 0)
    def _():
        o_ref[...] = jnp.zeros_like(o_ref)

    o_ref[...] += jnp.dot(h_ref[...], w_ref[...], preferred_element_type=jnp.float32)

    @pl.when(k == nk - 1)
    def _():
        o_ref[...] = jnp.maximum(o_ref[...] + b_ref[...], 0.0)


def _fc6(h0, W1, b1):
    n, K = h0.shape
    J = W1.shape[1]
    bk = 896
    return pl.pallas_call(
        _fc6_kernel,
        out_shape=jax.ShapeDtypeStruct((n, J), jnp.float32),
        grid=(K // bk,),
        in_specs=[
            pl.BlockSpec((n, bk), lambda k: (0, k)),
            pl.BlockSpec((bk, J), lambda k: (k, 0)),
            pl.BlockSpec((1, J), lambda k: (0, 0)),
        ],
        out_specs=pl.BlockSpec((n, J), lambda k: (0, 0)),
        compiler_params=pltpu.CompilerParams(
            dimension_semantics=("arbitrary",),
            vmem_limit_bytes=56 * 1024 * 1024,
        ),
        name="fc6",
    )(h0, W1, b1)


def _fc7_heads_kernel(h_ref, w2_ref, b2_ref, whc_ref, bhc_ref, o_ref):
    j = pl.program_id(0)
    nj = pl.num_programs(0)
    t = jnp.maximum(
        jnp.dot(h_ref[...], w2_ref[...], preferred_element_type=jnp.float32) + b2_ref[...], 0.0
    )

    @pl.when(j == 0)
    def _():
        o_ref[...] = bhc_ref[...] + jnp.zeros_like(o_ref)

    o_ref[...] += jnp.dot(t, whc_ref[...], preferred_element_type=jnp.float32)


def _fc7_heads(h1, W2, b2, whc, bhc):
    n, K = h1.shape
    M = whc.shape[1]
    bj = 512
    return pl.pallas_call(
        _fc7_heads_kernel,
        out_shape=jax.ShapeDtypeStruct((n, M), jnp.float32),
        grid=(K // bj,),
        in_specs=[
            pl.BlockSpec((n, K), lambda j: (0, 0)),
            pl.BlockSpec((K, bj), lambda j: (0, j)),
            pl.BlockSpec((1, bj), lambda j: (0, j)),
            pl.BlockSpec((bj, M), lambda j: (j, 0)),
            pl.BlockSpec((1, M), lambda j: (0, 0)),
        ],
        out_specs=pl.BlockSpec((n, M), lambda j: (0, 0)),
        compiler_params=pltpu.CompilerParams(
            dimension_semantics=("arbitrary",),
            vmem_limit_bytes=56 * 1024 * 1024,
        ),
        name="fc7_heads",
    )(h1, W2, b2, whc, bhc)


def kernel(x, rois, rois_index, W1, b1, W2, b2, Wb, bb, Wc, bc):
    B, C, H, W = x.shape
    n = rois.shape[0]
    pooled = _pool(x, rois, rois_index)
    h0 = pooled.transpose(0, 3, 1, 2).reshape(n, C * _P * _P)
    h1 = _fc6(h0, W1, b1.reshape(1, -1))
    whc = jnp.concatenate([Wb, Wc], axis=1)
    bhc = jnp.concatenate([bb, bc]).reshape(1, -1)
    heads = _fc7_heads(h1, W2, b2.reshape(1, -1), whc, bhc)
    nb = Wb.shape[1]
    return heads[:, :nb], heads[:, nb:]
